# Initial kernel scaffold; baseline (speedup 1.0000x reference)
import jax
import jax.numpy as jnp
from jax.experimental import pallas as pl


def kernel(inputs, word_table, pos_table):
    raise NotImplementedError("write your pallas kernel here")


# speedup vs baseline: 3.5135x; 3.5135x over previous
import functools

import jax
import jax.numpy as jnp
from jax import lax
from jax.experimental import pallas as pl
from jax.experimental.pallas import tpu as pltpu
from jax.experimental.pallas import tpu_sc as plsc

L16 = 16


def _make_sc_kernel(B, L, D, V):
    info = plsc.get_sparse_core_info()
    NC, NS = info.num_cores, info.num_subcores
    NW = NC * NS
    assert B % NW == 0
    ROWS_PER_W = B // NW
    CB = 4
    NCHUNK = ROWS_PER_W // CB
    CROWS = CB * L
    SUBS = [128] * (CROWS // 128)
    if CROWS % 128:
        SUBS.append(CROWS % 128)

    mesh = plsc.VectorSubcoreMesh(core_axis_name="c", subcore_axis_name="s")

    @functools.partial(
        pl.kernel,
        mesh=mesh,
        compiler_params=pltpu.CompilerParams(use_tc_tiling_on_sc=False),
        out_type=jax.ShapeDtypeStruct((B * L, D), jnp.float32),
        scratch_types=[
            pltpu.VMEM((L, D), jnp.float32),
            pltpu.VMEM((CROWS,), jnp.int32),
            pltpu.VMEM((CROWS, D), jnp.float32),
            pltpu.SemaphoreType.DMA,
        ],
    )
    def sc_kernel(idx_hbm, word_hbm, pos_hbm, out_hbm, pos_v, idx_v, rows_v, sem):
        wid = lax.axis_index("s") * NC + lax.axis_index("c")
        pltpu.sync_copy(pos_hbm, pos_v)
        base = wid * ROWS_PER_W

        def chunk_body(g, carry):
            flat0 = (base + g * CB) * L
            pltpu.sync_copy(idx_hbm.at[pl.ds(flat0, CROWS)], idx_v)
            cps = []
            off = 0
            for sz in SUBS:
                cps.append(
                    pltpu.async_copy(
                        word_hbm.at[idx_v.at[pl.ds(off, sz)]],
                        rows_v.at[pl.ds(off, sz)],
                        sem,
                    )
                )
                off += sz
            for cp in cps:
                cp.wait()

            def add_body(l, carry2):
                for rb in range(CB):
                    for c in range(D // L16):
                        pv = pos_v[l, pl.ds(c * L16, L16)]
                        plsc.addupdate(
                            rows_v.at[rb * L + l, pl.ds(c * L16, L16)], pv
                        )
                return carry2

            lax.fori_loop(0, L, add_body, 0)
            pltpu.sync_copy(rows_v, out_hbm.at[pl.ds(flat0, CROWS)])
            return carry

        lax.fori_loop(0, NCHUNK, chunk_body, 0)

    return sc_kernel


def kernel(inputs, word_table, pos_table):
    B, L = inputs.shape
    V, D = word_table.shape
    idx = inputs.reshape(B * L).astype(jnp.int32)
    sc = _make_sc_kernel(B, L, D, V)
    out = sc(idx, word_table, pos_table)
    return out.reshape(B, L, D)


# speedup vs baseline: 4.1706x; 1.1870x over previous
import functools

import jax
import jax.numpy as jnp
from jax import lax
from jax.experimental import pallas as pl
from jax.experimental.pallas import tpu as pltpu
from jax.experimental.pallas import tpu_sc as plsc

L16 = 16


def _make_sc_kernel(B, L, D, V):
    info = plsc.get_sparse_core_info()
    NC, NS = info.num_cores, info.num_subcores
    NW = NC * NS
    assert B % NW == 0
    ROWS_PER_W = B // NW
    CB = 2
    NCHUNK = ROWS_PER_W // CB
    CROWS = CB * L
    WROWS = ROWS_PER_W * L
    SUBS = [128] * (CROWS // 128)
    if CROWS % 128:
        SUBS.append(CROWS % 128)

    mesh = plsc.VectorSubcoreMesh(core_axis_name="c", subcore_axis_name="s")

    @functools.partial(
        pl.kernel,
        mesh=mesh,
        compiler_params=pltpu.CompilerParams(use_tc_tiling_on_sc=False),
        out_type=jax.ShapeDtypeStruct((B * L, D), jnp.float32),
        scratch_types=[
            pltpu.VMEM((L, D), jnp.float32),
            pltpu.VMEM((WROWS,), jnp.int32),
            pltpu.VMEM((CROWS, D), jnp.float32),
            pltpu.VMEM((CROWS, D), jnp.float32),
            pltpu.SemaphoreType.DMA,
            pltpu.SemaphoreType.DMA,
            pltpu.SemaphoreType.DMA,
            pltpu.SemaphoreType.DMA,
        ],
    )
    def sc_kernel(idx_hbm, word_hbm, pos_hbm, out_hbm,
                  pos_v, idx_v, rows0, rows1, gsem0, gsem1, osem0, osem1):
        rows = (rows0, rows1)
        gsem = (gsem0, gsem1)
        osem = (osem0, osem1)
        wid = lax.axis_index("s") * NC + lax.axis_index("c")
        wbase = wid * WROWS
        pltpu.sync_copy(pos_hbm, pos_v)
        pltpu.sync_copy(idx_hbm.at[pl.ds(wbase, WROWS)], idx_v)

        def issue_gathers(g, p):
            off = 0
            for sz in SUBS:
                pltpu.async_copy(
                    word_hbm.at[idx_v.at[pl.ds(g * CROWS + off, sz)]],
                    rows[p].at[pl.ds(off, sz)],
                    gsem[p],
                )
                off += sz

        def wait_gathers(p):
            pltpu.make_async_copy(
                out_hbm.at[pl.ds(0, CROWS)], rows[p], gsem[p]
            ).wait()

        def wait_writeback(p):
            pltpu.make_async_copy(
                rows[p], out_hbm.at[pl.ds(0, CROWS)], osem[p]
            ).wait()

        def add_and_flush(g, p):
            def add_body(l, carry2):
                for c in range(D // L16):
                    pv = pos_v[l, pl.ds(c * L16, L16)]
                    for rb in range(CB):
                        plsc.addupdate(
                            rows[p].at[rb * L + l, pl.ds(c * L16, L16)], pv
                        )
                return carry2

            lax.fori_loop(0, L, add_body, 0)
            pltpu.async_copy(
                rows[p], out_hbm.at[pl.ds(wbase + g * CROWS, CROWS)], osem[p]
            )

        issue_gathers(0, 0)

        def loop_body(j, carry):
            a = 2 * j
            @pl.when(j > 0)
            def _():
                wait_writeback(1)
            issue_gathers(a + 1, 1)
            wait_gathers(0)
            add_and_flush(a, 0)
            @pl.when(j < NCHUNK // 2 - 1)
            def _():
                wait_writeback(0)
                issue_gathers(a + 2, 0)
            wait_gathers(1)
            add_and_flush(a + 1, 1)
            return carry

        lax.fori_loop(0, NCHUNK // 2, loop_body, 0)
        wait_writeback(0)
        wait_writeback(1)

    return sc_kernel


def kernel(inputs, word_table, pos_table):
    B, L = inputs.shape
    V, D = word_table.shape
    idx = inputs.reshape(B * L).astype(jnp.int32)
    sc = _make_sc_kernel(B, L, D, V)
    out = sc(idx, word_table, pos_table)
    return out.reshape(B, L, D)


# speedup vs baseline: 4.1789x; 1.0020x over previous
import functools

import jax
import jax.numpy as jnp
from jax import lax
from jax.experimental import pallas as pl
from jax.experimental.pallas import tpu as pltpu
from jax.experimental.pallas import tpu_sc as plsc

L16 = 16


def _make_sc_kernel(B, L, D, V):
    info = plsc.get_sparse_core_info()
    NC, NS = info.num_cores, info.num_subcores
    NW = NC * NS
    assert B % NW == 0
    ROWS_PER_W = B // NW
    CB = 2
    NCHUNK = ROWS_PER_W // CB
    CROWS = CB * L
    WROWS = ROWS_PER_W * L
    SUBS = [128] * (CROWS // 128)
    if CROWS % 128:
        SUBS.append(CROWS % 128)

    mesh = plsc.VectorSubcoreMesh(core_axis_name="c", subcore_axis_name="s")

    @functools.partial(
        pl.kernel,
        mesh=mesh,
        compiler_params=pltpu.CompilerParams(use_tc_tiling_on_sc=False),
        out_type=jax.ShapeDtypeStruct((B, L, D), jnp.float32),
        scratch_types=[
            pltpu.VMEM((L, D), jnp.float32),
            pltpu.VMEM((WROWS,), jnp.int32),
            pltpu.VMEM((CROWS, D), jnp.float32),
            pltpu.VMEM((CROWS, D), jnp.float32),
            pltpu.SemaphoreType.DMA,
            pltpu.SemaphoreType.DMA,
            pltpu.SemaphoreType.DMA,
            pltpu.SemaphoreType.DMA,
        ],
    )
    def sc_kernel(idx_hbm, word_hbm, pos_hbm, out_hbm,
                  pos_v, idx_v, rows0, rows1, gsem0, gsem1, osem0, osem1):
        rows = (rows0, rows1)
        gsem = (gsem0, gsem1)
        osem = (osem0, osem1)
        wid = lax.axis_index("s") * NC + lax.axis_index("c")
        wbase = wid * WROWS
        pltpu.sync_copy(pos_hbm, pos_v)
        pltpu.sync_copy(idx_hbm.at[pl.ds(wbase, WROWS)], idx_v)

        def issue_gathers(g, p):
            off = 0
            for sz in SUBS:
                pltpu.async_copy(
                    word_hbm.at[idx_v.at[pl.ds(g * CROWS + off, sz)]],
                    rows[p].at[pl.ds(off, sz)],
                    gsem[p],
                )
                off += sz

        def wait_gathers(p):
            pltpu.make_async_copy(
                word_hbm.at[pl.ds(0, CROWS)], rows[p], gsem[p]
            ).wait()

        def wait_writeback(p):
            for rb in range(CB):
                pltpu.make_async_copy(
                    rows[p].at[pl.ds(rb * L, L)], out_hbm.at[0], osem[p]
                ).wait()

        def add_and_flush(g, p):
            def add_body(l, carry2):
                for c in range(D // L16):
                    pv = pos_v[l, pl.ds(c * L16, L16)]
                    for rb in range(CB):
                        plsc.addupdate(
                            rows[p].at[rb * L + l, pl.ds(c * L16, L16)], pv
                        )
                return carry2

            lax.fori_loop(0, L, add_body, 0)
            bb = wid * ROWS_PER_W + g * CB
            for rb in range(CB):
                pltpu.async_copy(
                    rows[p].at[pl.ds(rb * L, L)], out_hbm.at[bb + rb], osem[p]
                )

        issue_gathers(0, 0)

        def loop_body(j, carry):
            a = 2 * j
            @pl.when(j > 0)
            def _():
                wait_writeback(1)
            issue_gathers(a + 1, 1)
            wait_gathers(0)
            add_and_flush(a, 0)
            @pl.when(j < NCHUNK // 2 - 1)
            def _():
                wait_writeback(0)
                issue_gathers(a + 2, 0)
            wait_gathers(1)
            add_and_flush(a + 1, 1)
            return carry

        lax.fori_loop(0, NCHUNK // 2, loop_body, 0)
        wait_writeback(0)
        wait_writeback(1)

    return sc_kernel


def kernel(inputs, word_table, pos_table):
    B, L = inputs.shape
    V, D = word_table.shape
    idx = inputs.reshape(B * L).astype(jnp.int32)
    sc = _make_sc_kernel(B, L, D, V)
    return sc(idx, word_table, pos_table)
